# Initial kernel scaffold; baseline (speedup 1.0000x reference)
import jax
import jax.numpy as jnp
from jax.experimental import pallas as pl


def kernel(z, z_pos, codebook):
    raise NotImplementedError("write your pallas kernel here")


# speedup vs baseline: 2.7965x; 2.7965x over previous
import functools

import jax
import jax.numpy as jnp
from jax import lax
from jax.experimental import pallas as pl
from jax.experimental.pallas import tpu as pltpu
from jax.experimental.pallas import tpu_sc as plsc

_TN = 128


def _vq_body(z_ref, zp_ref, cb_ref, prob_ref, pprob_ref, idx_ref, csq_ref):
    i = pl.program_id(0)
    cb = cb_ref[...]

    @pl.when(i == 0)
    def _():
        csq_ref[...] = jnp.sum(cb * cb, axis=1)

    csq = csq_ref[...]
    k = cb.shape[0]

    def dist_softmax(x):
        xsq = jnp.sum(x * x, axis=1, keepdims=True)
        mm = lax.dot_general(x, cb, (((1,), (1,)), ((), ())),
                             preferred_element_type=jnp.float32)
        dist = (xsq + csq[None, :]) - 2.0 * mm
        minval = jnp.min(dist, axis=1, keepdims=True)
        e = jnp.exp(minval - dist)
        s = jnp.sum(e, axis=1, keepdims=True)
        return dist, minval, e * (1.0 / s)

    dist, minval, prob = dist_softmax(z_ref[...])
    prob_ref[...] = prob
    iota = lax.broadcasted_iota(jnp.int32, dist.shape, 1)
    idx_ref[...] = jnp.min(jnp.where(dist == minval, iota, jnp.int32(k)),
                           axis=1)

    _, _, pprob = dist_softmax(zp_ref[...])
    pprob_ref[...] = pprob


def _vq_pallas(z_flat, zp_flat, codebook):
    n, d = z_flat.shape
    k = codebook.shape[0]
    grid = (n // _TN,)
    return pl.pallas_call(
        _vq_body,
        grid=grid,
        in_specs=[
            pl.BlockSpec((_TN, d), lambda i: (i, 0)),
            pl.BlockSpec((_TN, d), lambda i: (i, 0)),
            pl.BlockSpec((k, d), lambda i: (0, 0)),
        ],
        out_specs=[
            pl.BlockSpec((_TN, k), lambda i: (i, 0)),
            pl.BlockSpec((_TN, k), lambda i: (i, 0)),
            pl.BlockSpec((_TN,), lambda i: (i,)),
        ],
        out_shape=[
            jax.ShapeDtypeStruct((n, k), jnp.float32),
            jax.ShapeDtypeStruct((n, k), jnp.float32),
            jax.ShapeDtypeStruct((n,), jnp.int32),
        ],
        scratch_shapes=[pltpu.VMEM((k,), jnp.float32)],
        compiler_params=pltpu.CompilerParams(
            dimension_semantics=("arbitrary",)),
    )(z_flat, zp_flat, codebook)


def _sc_gather(table, idx):
    n = idx.shape[0]
    d = table.shape[1]
    info = plsc.get_sparse_core_info()
    nw = info.num_cores * info.num_subcores
    b_per_w = n // nw
    ch = 96
    nch = b_per_w // ch
    mesh = plsc.VectorSubcoreMesh(core_axis_name="c", subcore_axis_name="s")

    @functools.partial(
        pl.kernel, mesh=mesh,
        out_type=jax.ShapeDtypeStruct((n, d), jnp.float32),
        scratch_types=[
            pltpu.VMEM((ch,), jnp.int32),
            pltpu.VMEM((ch, d), jnp.float32),
            pltpu.SemaphoreType.DMA,
        ],
    )
    def gath(table_hbm, idx_hbm, out_hbm, idx_v, rows_v, sem):
        wid = lax.axis_index("s") * info.num_cores + lax.axis_index("c")
        base = wid * b_per_w
        for j in range(nch):
            off = base + j * ch
            pltpu.sync_copy(idx_hbm.at[pl.ds(off, ch)], idx_v)
            pltpu.async_copy(table_hbm.at[idx_v], rows_v, sem).wait()
            pltpu.sync_copy(rows_v, out_hbm.at[pl.ds(off, ch)])

    return gath(table, idx)


def kernel(z, z_pos, codebook):
    b, d, h, w = z.shape
    n = b * h * w
    z_flat = jnp.transpose(z, (0, 2, 3, 1)).reshape(n, d)
    zp_flat = jnp.transpose(z_pos, (0, 2, 3, 1)).reshape(n, d)
    prob, pprob, idx = _vq_pallas(z_flat, zp_flat, codebook)
    zq_flat = _sc_gather(codebook, idx)
    z_q = jnp.transpose(zq_flat.reshape(b, h, w, d), (0, 3, 1, 2))
    return z_q, idx, prob, pprob


# speedup vs baseline: 2.9568x; 1.0573x over previous
import functools

import jax
import jax.numpy as jnp
from jax import lax
from jax.experimental import pallas as pl
from jax.experimental.pallas import tpu as pltpu
from jax.experimental.pallas import tpu_sc as plsc

_TN = 128


def _csq_body(cb_ref, csq_ref):
    cb = cb_ref[...]
    csq_ref[...] = jnp.sum(cb * cb, axis=1)[None, :]


def _csq_pallas(codebook):
    k, d = codebook.shape
    return pl.pallas_call(
        _csq_body,
        out_shape=jax.ShapeDtypeStruct((1, k), jnp.float32),
    )(codebook)


def _vq_body(z_ref, zp_ref, cb_ref, csq_ref, prob_ref, pprob_ref, idx_ref):
    csq = csq_ref[...]
    k = cb_ref.shape[0]

    def dist_softmax(x):
        xsq = jnp.sum(x * x, axis=1, keepdims=True)
        mm = lax.dot_general(x, cb_ref[...], (((1,), (1,)), ((), ())),
                             preferred_element_type=jnp.float32)
        dist = (xsq + csq) - 2.0 * mm
        minval = jnp.min(dist, axis=1, keepdims=True)
        e = jnp.exp(minval - dist)
        s = jnp.sum(e, axis=1, keepdims=True)
        return dist, minval, e * (1.0 / s)

    dist, minval, prob = dist_softmax(z_ref[...])
    prob_ref[...] = prob
    iota = lax.broadcasted_iota(jnp.int32, dist.shape, 1)
    idx_ref[...] = jnp.min(jnp.where(dist == minval, iota, jnp.int32(k)),
                           axis=1)

    _, _, pprob = dist_softmax(zp_ref[...])
    pprob_ref[...] = pprob


def _vq_pallas(z_flat, zp_flat, codebook, csq):
    n, d = z_flat.shape
    k = codebook.shape[0]
    grid = (n // _TN,)
    return pl.pallas_call(
        _vq_body,
        grid=grid,
        in_specs=[
            pl.BlockSpec((_TN, d), lambda i: (i, 0)),
            pl.BlockSpec((_TN, d), lambda i: (i, 0)),
            pl.BlockSpec((k, d), lambda i: (0, 0)),
            pl.BlockSpec((1, k), lambda i: (0, 0)),
        ],
        out_specs=[
            pl.BlockSpec((_TN, k), lambda i: (i, 0)),
            pl.BlockSpec((_TN, k), lambda i: (i, 0)),
            pl.BlockSpec((_TN,), lambda i: (i,)),
        ],
        out_shape=[
            jax.ShapeDtypeStruct((n, k), jnp.float32),
            jax.ShapeDtypeStruct((n, k), jnp.float32),
            jax.ShapeDtypeStruct((n,), jnp.int32),
        ],
        compiler_params=pltpu.CompilerParams(
            dimension_semantics=("arbitrary",)),
    )(z_flat, zp_flat, codebook, csq)


def _sc_gather(table, idx):
    n = idx.shape[0]
    d = table.shape[1]
    info = plsc.get_sparse_core_info()
    nw = info.num_cores * info.num_subcores
    b_per_w = n // nw
    ch = 96
    nch = b_per_w // ch
    mesh = plsc.VectorSubcoreMesh(core_axis_name="c", subcore_axis_name="s")

    @functools.partial(
        pl.kernel, mesh=mesh,
        out_type=jax.ShapeDtypeStruct((n, d), jnp.float32),
        scratch_types=[
            pltpu.VMEM((ch,), jnp.int32),
            pltpu.VMEM((ch, d), jnp.float32),
            pltpu.SemaphoreType.DMA,
        ],
    )
    def gath(table_hbm, idx_hbm, out_hbm, idx_v, rows_v, sem):
        wid = lax.axis_index("s") * info.num_cores + lax.axis_index("c")
        base = wid * b_per_w
        for j in range(nch):
            off = base + j * ch
            pltpu.sync_copy(idx_hbm.at[pl.ds(off, ch)], idx_v)
            pltpu.async_copy(table_hbm.at[idx_v], rows_v, sem).wait()
            pltpu.sync_copy(rows_v, out_hbm.at[pl.ds(off, ch)])

    return gath(table, idx)


def kernel(z, z_pos, codebook):
    b, d, h, w = z.shape
    n = b * h * w
    z_flat = jnp.transpose(z, (0, 2, 3, 1)).reshape(n, d)
    zp_flat = jnp.transpose(z_pos, (0, 2, 3, 1)).reshape(n, d)
    csq = _csq_pallas(codebook)
    prob, pprob, idx = _vq_pallas(z_flat, zp_flat, codebook, csq)
    zq_flat = _sc_gather(codebook, idx)
    z_q = jnp.transpose(zq_flat.reshape(b, h, w, d), (0, 3, 1, 2))
    return z_q, idx, prob, pprob


# speedup vs baseline: 3.7419x; 1.2655x over previous
import functools

import jax
import jax.numpy as jnp
from jax import lax
from jax.experimental import pallas as pl
from jax.experimental.pallas import tpu as pltpu
from jax.experimental.pallas import tpu_sc as plsc

_TN = 128


def _csq_body(cb_ref, csq_ref):
    cb = cb_ref[...]
    csq_ref[...] = jnp.sum(cb * cb, axis=1)[None, :]


def _csq_pallas(codebook):
    k, d = codebook.shape
    return pl.pallas_call(
        _csq_body,
        out_shape=jax.ShapeDtypeStruct((1, k), jnp.float32),
    )(codebook)


def _vq_body(z_ref, zp_ref, cb_ref, csq_ref, prob_ref, pprob_ref, idx_ref):
    csq = csq_ref[...]
    tn = z_ref.shape[0]

    x = z_ref[...]
    xp = zp_ref[...]
    xcat = jnp.concatenate([x, xp], axis=0) * (-2.0)
    mm2 = lax.dot_general(xcat, cb_ref[...], (((1,), (1,)), ((), ())),
                          preferred_element_type=jnp.float32)

    def softmax_part(v, mm2_part):
        vsq = jnp.sum(v * v, axis=1, keepdims=True)
        dist = (vsq + csq) + mm2_part
        minval = jnp.min(dist, axis=1, keepdims=True)
        e = jnp.exp(minval - dist)
        s = jnp.sum(e, axis=1, keepdims=True)
        return dist, minval, e * (1.0 / s)

    dist, minval, prob = softmax_part(x, mm2[:tn])
    prob_ref[...] = prob
    iota = lax.broadcasted_iota(jnp.int32, dist.shape, 1)
    idx_ref[...] = jnp.min(jnp.where(dist == minval, iota,
                                     jnp.int32(dist.shape[1])), axis=1)

    _, _, pprob = softmax_part(xp, mm2[tn:])
    pprob_ref[...] = pprob


def _vq_pallas(z_flat, zp_flat, codebook, csq):
    n, d = z_flat.shape
    k = codebook.shape[0]
    grid = (n // _TN,)
    return pl.pallas_call(
        _vq_body,
        grid=grid,
        in_specs=[
            pl.BlockSpec((_TN, d), lambda i: (i, 0)),
            pl.BlockSpec((_TN, d), lambda i: (i, 0)),
            pl.BlockSpec((k, d), lambda i: (0, 0)),
            pl.BlockSpec((1, k), lambda i: (0, 0)),
        ],
        out_specs=[
            pl.BlockSpec((_TN, k), lambda i: (i, 0)),
            pl.BlockSpec((_TN, k), lambda i: (i, 0)),
            pl.BlockSpec((_TN,), lambda i: (i,)),
        ],
        out_shape=[
            jax.ShapeDtypeStruct((n, k), jnp.float32),
            jax.ShapeDtypeStruct((n, k), jnp.float32),
            jax.ShapeDtypeStruct((n,), jnp.int32),
        ],
        compiler_params=pltpu.CompilerParams(
            dimension_semantics=("arbitrary",)),
    )(z_flat, zp_flat, codebook, csq)


def _sc_gather(table, idx):
    n = idx.shape[0]
    d = table.shape[1]
    info = plsc.get_sparse_core_info()
    nw = info.num_cores * info.num_subcores
    b_per_w = n // nw
    ch = 96
    nch = b_per_w // ch
    mesh = plsc.VectorSubcoreMesh(core_axis_name="c", subcore_axis_name="s")

    @functools.partial(
        pl.kernel, mesh=mesh,
        out_type=jax.ShapeDtypeStruct((n, d), jnp.float32),
        scratch_types=[
            pltpu.VMEM((ch,), jnp.int32),
            pltpu.VMEM((ch, d), jnp.float32),
            pltpu.SemaphoreType.DMA,
        ],
    )
    def gath(table_hbm, idx_hbm, out_hbm, idx_v, rows_v, sem):
        wid = lax.axis_index("s") * info.num_cores + lax.axis_index("c")
        base = wid * b_per_w
        for j in range(nch):
            off = base + j * ch
            pltpu.sync_copy(idx_hbm.at[pl.ds(off, ch)], idx_v)
            pltpu.async_copy(table_hbm.at[idx_v], rows_v, sem).wait()
            pltpu.sync_copy(rows_v, out_hbm.at[pl.ds(off, ch)])

    return gath(table, idx)


def kernel(z, z_pos, codebook):
    b, d, h, w = z.shape
    n = b * h * w
    z_flat = jnp.transpose(z, (0, 2, 3, 1)).reshape(n, d)
    zp_flat = jnp.transpose(z_pos, (0, 2, 3, 1)).reshape(n, d)
    csq = _csq_pallas(codebook)
    prob, pprob, idx = _vq_pallas(z_flat, zp_flat, codebook, csq)
    zq_flat = _sc_gather(codebook, idx)
    z_q = jnp.transpose(zq_flat.reshape(b, h, w, d), (0, 3, 1, 2))
    return z_q, idx, prob, pprob


# speedup vs baseline: 3.8427x; 1.0269x over previous
import functools

import jax
import jax.numpy as jnp
from jax import lax
from jax.experimental import pallas as pl
from jax.experimental.pallas import tpu as pltpu
from jax.experimental.pallas import tpu_sc as plsc

_TN = 128


def _csq_body(cb_ref, csq_ref):
    cb = cb_ref[...]
    csq_ref[...] = jnp.sum(cb * cb, axis=1)[None, :]


def _csq_pallas(codebook):
    k, d = codebook.shape
    return pl.pallas_call(
        _csq_body,
        out_shape=jax.ShapeDtypeStruct((1, k), jnp.float32),
    )(codebook)


def _vq_body(z_ref, zp_ref, cb_ref, csq_ref, iota_ref,
             prob_ref, pprob_ref, idx_ref):
    csq = csq_ref[...]
    tn = z_ref.shape[0]

    x = z_ref[...]
    xp = zp_ref[...]
    xcat = jnp.concatenate([x, xp], axis=0) * (-2.0)
    mm2 = lax.dot_general(xcat, cb_ref[...], (((1,), (1,)), ((), ())),
                          preferred_element_type=jnp.float32)

    def softmax_part(v, mm2_part):
        vsq = jnp.sum(v * v, axis=1, keepdims=True)
        dist = (vsq + csq) + mm2_part
        minval = jnp.min(dist, axis=1, keepdims=True)
        e = jnp.exp(minval - dist)
        s = jnp.sum(e, axis=1, keepdims=True)
        return dist, minval, e * (1.0 / s)

    dist, minval, prob = softmax_part(x, mm2[:tn])
    prob_ref[...] = prob
    idxf = jnp.min(jnp.where(dist == minval, iota_ref[...],
                             jnp.float32(2 ** 24)), axis=1)
    idx_ref[...] = idxf.astype(jnp.int32)

    _, _, pprob = softmax_part(xp, mm2[tn:])
    pprob_ref[...] = pprob


def _vq_pallas(z_flat, zp_flat, codebook, csq):
    n, d = z_flat.shape
    k = codebook.shape[0]
    grid = (n // _TN,)
    iota_row = jnp.arange(k, dtype=jnp.float32)[None, :]
    return pl.pallas_call(
        _vq_body,
        grid=grid,
        in_specs=[
            pl.BlockSpec((_TN, d), lambda i: (i, 0)),
            pl.BlockSpec((_TN, d), lambda i: (i, 0)),
            pl.BlockSpec((k, d), lambda i: (0, 0)),
            pl.BlockSpec((1, k), lambda i: (0, 0)),
            pl.BlockSpec((1, k), lambda i: (0, 0)),
        ],
        out_specs=[
            pl.BlockSpec((_TN, k), lambda i: (i, 0)),
            pl.BlockSpec((_TN, k), lambda i: (i, 0)),
            pl.BlockSpec((_TN,), lambda i: (i,)),
        ],
        out_shape=[
            jax.ShapeDtypeStruct((n, k), jnp.float32),
            jax.ShapeDtypeStruct((n, k), jnp.float32),
            jax.ShapeDtypeStruct((n,), jnp.int32),
        ],
        compiler_params=pltpu.CompilerParams(
            dimension_semantics=("arbitrary",)),
    )(z_flat, zp_flat, codebook, csq, iota_row)


def _sc_gather(table, idx):
    n = idx.shape[0]
    d = table.shape[1]
    info = plsc.get_sparse_core_info()
    nw = info.num_cores * info.num_subcores
    b_per_w = n // nw
    ch = 96
    nch = b_per_w // ch
    mesh = plsc.VectorSubcoreMesh(core_axis_name="c", subcore_axis_name="s")

    @functools.partial(
        pl.kernel, mesh=mesh,
        out_type=jax.ShapeDtypeStruct((n, d), jnp.float32),
        scratch_types=[
            pltpu.VMEM((ch,), jnp.int32),
            pltpu.VMEM((ch, d), jnp.float32),
            pltpu.SemaphoreType.DMA,
        ],
    )
    def gath(table_hbm, idx_hbm, out_hbm, idx_v, rows_v, sem):
        wid = lax.axis_index("s") * info.num_cores + lax.axis_index("c")
        base = wid * b_per_w
        for j in range(nch):
            off = base + j * ch
            pltpu.sync_copy(idx_hbm.at[pl.ds(off, ch)], idx_v)
            pltpu.async_copy(table_hbm.at[idx_v], rows_v, sem).wait()
            pltpu.sync_copy(rows_v, out_hbm.at[pl.ds(off, ch)])

    return gath(table, idx)


def kernel(z, z_pos, codebook):
    b, d, h, w = z.shape
    n = b * h * w
    z_flat = jnp.transpose(z, (0, 2, 3, 1)).reshape(n, d)
    zp_flat = jnp.transpose(z_pos, (0, 2, 3, 1)).reshape(n, d)
    csq = _csq_pallas(codebook)
    prob, pprob, idx = _vq_pallas(z_flat, zp_flat, codebook, csq)
    zq_flat = _sc_gather(codebook, idx)
    z_q = jnp.transpose(zq_flat.reshape(b, h, w, d), (0, 3, 1, 2))
    return z_q, idx, prob, pprob


# speedup vs baseline: 3.8473x; 1.0012x over previous
import functools

import jax
import jax.numpy as jnp
from jax import lax
from jax.experimental import pallas as pl
from jax.experimental.pallas import tpu as pltpu
from jax.experimental.pallas import tpu_sc as plsc

_TN = 128


def _csq_body(cb_ref, csq_ref):
    cb = cb_ref[...]
    csq_ref[...] = jnp.sum(cb * cb, axis=1)[None, :]


def _csq_pallas(codebook):
    k, d = codebook.shape
    return pl.pallas_call(
        _csq_body,
        out_shape=jax.ShapeDtypeStruct((1, k), jnp.float32),
    )(codebook)


def _vq_body(z_ref, zp_ref, cb_ref, csq_ref, iota_ref,
             prob_ref, pprob_ref, idx_ref):
    csq = csq_ref[...]
    tn = z_ref.shape[0]

    x = z_ref[...]
    xp = zp_ref[...]
    xcat = jnp.concatenate([x, xp], axis=0) * (-2.0)
    mm2 = lax.dot_general(xcat, cb_ref[...], (((1,), (1,)), ((), ())),
                          preferred_element_type=jnp.float32)

    vsq = jnp.sum(x * x, axis=1, keepdims=True)
    dist = (vsq + csq) + mm2[:tn]
    minval = jnp.min(dist, axis=1, keepdims=True)
    t = minval - dist
    e = jnp.exp(t)
    s = jnp.sum(e, axis=1, keepdims=True)
    prob_ref[...] = e * (1.0 / s)
    idxf = jnp.min(jnp.where(t == 0.0, iota_ref[...],
                             jnp.float32(2 ** 24)), axis=1)
    idx_ref[...] = idxf.astype(jnp.int32)

    dist_p = csq + mm2[tn:]
    minval_p = jnp.min(dist_p, axis=1, keepdims=True)
    e_p = jnp.exp(minval_p - dist_p)
    s_p = jnp.sum(e_p, axis=1, keepdims=True)
    pprob_ref[...] = e_p * (1.0 / s_p)


def _vq_pallas(z_flat, zp_flat, codebook, csq):
    n, d = z_flat.shape
    k = codebook.shape[0]
    grid = (n // _TN,)
    iota_row = jnp.arange(k, dtype=jnp.float32)[None, :]
    return pl.pallas_call(
        _vq_body,
        grid=grid,
        in_specs=[
            pl.BlockSpec((_TN, d), lambda i: (i, 0)),
            pl.BlockSpec((_TN, d), lambda i: (i, 0)),
            pl.BlockSpec((k, d), lambda i: (0, 0)),
            pl.BlockSpec((1, k), lambda i: (0, 0)),
            pl.BlockSpec((1, k), lambda i: (0, 0)),
        ],
        out_specs=[
            pl.BlockSpec((_TN, k), lambda i: (i, 0)),
            pl.BlockSpec((_TN, k), lambda i: (i, 0)),
            pl.BlockSpec((_TN,), lambda i: (i,)),
        ],
        out_shape=[
            jax.ShapeDtypeStruct((n, k), jnp.float32),
            jax.ShapeDtypeStruct((n, k), jnp.float32),
            jax.ShapeDtypeStruct((n,), jnp.int32),
        ],
        compiler_params=pltpu.CompilerParams(
            dimension_semantics=("arbitrary",)),
    )(z_flat, zp_flat, codebook, csq, iota_row)


def _sc_gather(table, idx):
    n = idx.shape[0]
    d = table.shape[1]
    info = plsc.get_sparse_core_info()
    nw = info.num_cores * info.num_subcores
    b_per_w = n // nw
    ch = 96
    nch = b_per_w // ch
    mesh = plsc.VectorSubcoreMesh(core_axis_name="c", subcore_axis_name="s")

    @functools.partial(
        pl.kernel, mesh=mesh,
        out_type=jax.ShapeDtypeStruct((n, d), jnp.float32),
        scratch_types=[
            pltpu.VMEM((ch,), jnp.int32),
            pltpu.VMEM((ch, d), jnp.float32),
            pltpu.SemaphoreType.DMA,
        ],
    )
    def gath(table_hbm, idx_hbm, out_hbm, idx_v, rows_v, sem):
        wid = lax.axis_index("s") * info.num_cores + lax.axis_index("c")
        base = wid * b_per_w
        for j in range(nch):
            off = base + j * ch
            pltpu.sync_copy(idx_hbm.at[pl.ds(off, ch)], idx_v)
            pltpu.async_copy(table_hbm.at[idx_v], rows_v, sem).wait()
            pltpu.sync_copy(rows_v, out_hbm.at[pl.ds(off, ch)])

    return gath(table, idx)


def kernel(z, z_pos, codebook):
    b, d, h, w = z.shape
    n = b * h * w
    z_flat = jnp.transpose(z, (0, 2, 3, 1)).reshape(n, d)
    zp_flat = jnp.transpose(z_pos, (0, 2, 3, 1)).reshape(n, d)
    csq = _csq_pallas(codebook)
    prob, pprob, idx = _vq_pallas(z_flat, zp_flat, codebook, csq)
    zq_flat = _sc_gather(codebook, idx)
    z_q = jnp.transpose(zq_flat.reshape(b, h, w, d), (0, 3, 1, 2))
    return z_q, idx, prob, pprob


# speedup vs baseline: 4.0862x; 1.0621x over previous
import functools

import jax
import jax.numpy as jnp
from jax import lax
from jax.experimental import pallas as pl
from jax.experimental.pallas import tpu as pltpu
from jax.experimental.pallas import tpu_sc as plsc

_TN = 256


def _csq_body(cb_ref, csq_ref):
    cb = cb_ref[...]
    csq_ref[...] = jnp.sum(cb * cb, axis=1)[None, :]


def _csq_pallas(codebook):
    k, d = codebook.shape
    return pl.pallas_call(
        _csq_body,
        out_shape=jax.ShapeDtypeStruct((1, k), jnp.float32),
    )(codebook)


def _vq_body(z_ref, zp_ref, cb_ref, csq_ref, iota_ref,
             prob_ref, pprob_ref, idx_ref):
    csq = csq_ref[...]
    tn = z_ref.shape[0]

    x = z_ref[...]
    xp = zp_ref[...]
    xcat = jnp.concatenate([x, xp], axis=0) * (-2.0)
    mm2 = lax.dot_general(xcat, cb_ref[...], (((1,), (1,)), ((), ())),
                          preferred_element_type=jnp.float32)

    vsq = jnp.sum(x * x, axis=1, keepdims=True)
    dist = (vsq + csq) + mm2[:tn]
    minval = jnp.min(dist, axis=1, keepdims=True)
    t = minval - dist
    e = jnp.exp(t)
    s = jnp.sum(e, axis=1, keepdims=True)
    prob_ref[...] = e * (1.0 / s)
    idxf = jnp.min(jnp.where(t == 0.0, iota_ref[...],
                             jnp.float32(2 ** 24)), axis=1)
    idx_ref[...] = idxf.astype(jnp.int32)

    dist_p = csq + mm2[tn:]
    minval_p = jnp.min(dist_p, axis=1, keepdims=True)
    e_p = jnp.exp(minval_p - dist_p)
    s_p = jnp.sum(e_p, axis=1, keepdims=True)
    pprob_ref[...] = e_p * (1.0 / s_p)


def _vq_pallas(z_flat, zp_flat, codebook, csq):
    n, d = z_flat.shape
    k = codebook.shape[0]
    grid = (n // _TN,)
    iota_row = jnp.arange(k, dtype=jnp.float32)[None, :]
    return pl.pallas_call(
        _vq_body,
        grid=grid,
        in_specs=[
            pl.BlockSpec((_TN, d), lambda i: (i, 0)),
            pl.BlockSpec((_TN, d), lambda i: (i, 0)),
            pl.BlockSpec((k, d), lambda i: (0, 0)),
            pl.BlockSpec((1, k), lambda i: (0, 0)),
            pl.BlockSpec((1, k), lambda i: (0, 0)),
        ],
        out_specs=[
            pl.BlockSpec((_TN, k), lambda i: (i, 0)),
            pl.BlockSpec((_TN, k), lambda i: (i, 0)),
            pl.BlockSpec((_TN,), lambda i: (i,)),
        ],
        out_shape=[
            jax.ShapeDtypeStruct((n, k), jnp.float32),
            jax.ShapeDtypeStruct((n, k), jnp.float32),
            jax.ShapeDtypeStruct((n,), jnp.int32),
        ],
        compiler_params=pltpu.CompilerParams(
            dimension_semantics=("arbitrary",)),
    )(z_flat, zp_flat, codebook, csq, iota_row)


def _sc_gather(table, idx):
    n = idx.shape[0]
    d = table.shape[1]
    info = plsc.get_sparse_core_info()
    nw = info.num_cores * info.num_subcores
    b_per_w = n // nw
    ch = 96
    nch = b_per_w // ch
    mesh = plsc.VectorSubcoreMesh(core_axis_name="c", subcore_axis_name="s")

    @functools.partial(
        pl.kernel, mesh=mesh,
        out_type=jax.ShapeDtypeStruct((n, d), jnp.float32),
        scratch_types=[
            pltpu.VMEM((ch,), jnp.int32),
            pltpu.VMEM((ch, d), jnp.float32),
            pltpu.SemaphoreType.DMA,
        ],
    )
    def gath(table_hbm, idx_hbm, out_hbm, idx_v, rows_v, sem):
        wid = lax.axis_index("s") * info.num_cores + lax.axis_index("c")
        base = wid * b_per_w
        for j in range(nch):
            off = base + j * ch
            pltpu.sync_copy(idx_hbm.at[pl.ds(off, ch)], idx_v)
            pltpu.async_copy(table_hbm.at[idx_v], rows_v, sem).wait()
            pltpu.sync_copy(rows_v, out_hbm.at[pl.ds(off, ch)])

    return gath(table, idx)


def kernel(z, z_pos, codebook):
    b, d, h, w = z.shape
    n = b * h * w
    z_flat = jnp.transpose(z, (0, 2, 3, 1)).reshape(n, d)
    zp_flat = jnp.transpose(z_pos, (0, 2, 3, 1)).reshape(n, d)
    csq = _csq_pallas(codebook)
    prob, pprob, idx = _vq_pallas(z_flat, zp_flat, codebook, csq)
    zq_flat = _sc_gather(codebook, idx)
    z_q = jnp.transpose(zq_flat.reshape(b, h, w, d), (0, 3, 1, 2))
    return z_q, idx, prob, pprob


# speedup vs baseline: 4.0881x; 1.0005x over previous
import functools

import jax
import jax.numpy as jnp
from jax import lax
from jax.experimental import pallas as pl
from jax.experimental.pallas import tpu as pltpu
from jax.experimental.pallas import tpu_sc as plsc

_TN = 256


def _csq_body(cb_ref, csq_ref):
    cb = cb_ref[...]
    csq_ref[...] = jnp.sum(cb * cb, axis=1)[None, :]


def _csq_pallas(codebook):
    k, d = codebook.shape
    return pl.pallas_call(
        _csq_body,
        out_shape=jax.ShapeDtypeStruct((1, k), jnp.float32),
    )(codebook)


def _vq_body(z_ref, zp_ref, cb_ref, csq_ref, iota_ref,
             prob_ref, pprob_ref, idx_ref):
    csq = csq_ref[...]
    tn = z_ref.shape[0]

    x = z_ref[...]
    xp = zp_ref[...]
    xcat = jnp.concatenate([x, xp], axis=0) * (-2.0)
    mm2 = lax.dot_general(xcat, cb_ref[...], (((1,), (1,)), ((), ())),
                          preferred_element_type=jnp.float32)

    vsq = jnp.sum(x * x, axis=1, keepdims=True)
    dist = (vsq + csq) + mm2[:tn]
    minval = jnp.min(dist, axis=1, keepdims=True)
    t = minval - dist
    e = jnp.exp(t)
    s = jnp.sum(e, axis=1, keepdims=True)
    prob_ref[...] = e * (1.0 / s)
    idxf = jnp.min(jnp.where(t == 0.0, iota_ref[...],
                             jnp.float32(2 ** 24)), axis=1)
    idx_ref[...] = idxf.astype(jnp.int32)[None, None, :]

    dist_p = csq + mm2[tn:]
    minval_p = jnp.min(dist_p, axis=1, keepdims=True)
    e_p = jnp.exp(minval_p - dist_p)
    s_p = jnp.sum(e_p, axis=1, keepdims=True)
    pprob_ref[...] = e_p * (1.0 / s_p)


def _vq_pallas(z_flat, zp_flat, codebook, csq):
    n, d = z_flat.shape
    k = codebook.shape[0]
    grid = (n // _TN,)
    iota_row = jnp.arange(k, dtype=jnp.float32)[None, :]
    return pl.pallas_call(
        _vq_body,
        grid=grid,
        in_specs=[
            pl.BlockSpec((_TN, d), lambda i: (i, 0)),
            pl.BlockSpec((_TN, d), lambda i: (i, 0)),
            pl.BlockSpec((k, d), lambda i: (0, 0)),
            pl.BlockSpec((1, k), lambda i: (0, 0)),
            pl.BlockSpec((1, k), lambda i: (0, 0)),
        ],
        out_specs=[
            pl.BlockSpec((_TN, k), lambda i: (i, 0)),
            pl.BlockSpec((_TN, k), lambda i: (i, 0)),
            pl.BlockSpec((1, 1, _TN), lambda i: (i, 0, 0)),
        ],
        out_shape=[
            jax.ShapeDtypeStruct((n, k), jnp.float32),
            jax.ShapeDtypeStruct((n, k), jnp.float32),
            jax.ShapeDtypeStruct((n // _TN, 1, _TN), jnp.int32),
        ],
        compiler_params=pltpu.CompilerParams(
            dimension_semantics=("arbitrary",)),
    )(z_flat, zp_flat, codebook, csq, iota_row)


def _sc_gather(table, idx):
    n = idx.shape[0]
    d = table.shape[1]
    info = plsc.get_sparse_core_info()
    nw = info.num_cores * info.num_subcores
    b_per_w = n // nw
    ch = 96
    nch = b_per_w // ch
    mesh = plsc.VectorSubcoreMesh(core_axis_name="c", subcore_axis_name="s")

    @functools.partial(
        pl.kernel, mesh=mesh,
        out_type=jax.ShapeDtypeStruct((n, d), jnp.float32),
        scratch_types=[
            pltpu.VMEM((ch,), jnp.int32),
            pltpu.VMEM((ch, d), jnp.float32),
            pltpu.SemaphoreType.DMA,
        ],
    )
    def gath(table_hbm, idx_hbm, out_hbm, idx_v, rows_v, sem):
        wid = lax.axis_index("s") * info.num_cores + lax.axis_index("c")
        base = wid * b_per_w
        for j in range(nch):
            off = base + j * ch
            pltpu.sync_copy(idx_hbm.at[pl.ds(off, ch)], idx_v)
            pltpu.async_copy(table_hbm.at[idx_v], rows_v, sem).wait()
            pltpu.sync_copy(rows_v, out_hbm.at[pl.ds(off, ch)])

    return gath(table, idx)


def kernel(z, z_pos, codebook):
    b, d, h, w = z.shape
    n = b * h * w
    z_flat = jnp.transpose(z, (0, 2, 3, 1)).reshape(n, d)
    zp_flat = jnp.transpose(z_pos, (0, 2, 3, 1)).reshape(n, d)
    csq = _csq_pallas(codebook)
    prob, pprob, idx = _vq_pallas(z_flat, zp_flat, codebook, csq)
    idx = idx.reshape(n)
    zq_flat = _sc_gather(codebook, idx)
    z_q = jnp.transpose(zq_flat.reshape(b, h, w, d), (0, 3, 1, 2))
    return z_q, idx, prob, pprob


# speedup vs baseline: 4.1151x; 1.0066x over previous
import functools

import jax
import jax.numpy as jnp
from jax import lax
from jax.experimental import pallas as pl
from jax.experimental.pallas import tpu as pltpu
from jax.experimental.pallas import tpu_sc as plsc

_TN = 256


def _csq_body(cb_ref, csq_ref):
    cb = cb_ref[...]
    csq_ref[...] = jnp.sum(cb * cb, axis=1)[None, :]


def _csq_pallas(codebook):
    k, d = codebook.shape
    return pl.pallas_call(
        _csq_body,
        out_shape=jax.ShapeDtypeStruct((1, k), jnp.float32),
    )(codebook)


def _vq_body(z_ref, zp_ref, cb_ref, csq_ref, iota_ref,
             prob_ref, pprob_ref, idx_ref):
    csq = csq_ref[...]
    tn = z_ref.shape[0]

    x = z_ref[...]
    xp = zp_ref[...]
    xcat = jnp.concatenate([x, xp], axis=0) * (-2.0)
    mm2 = lax.dot_general(xcat, cb_ref[...], (((1,), (1,)), ((), ())),
                          preferred_element_type=jnp.float32)

    vsq = jnp.sum(x * x, axis=1, keepdims=True)
    dist = (vsq + csq) + mm2[:tn]
    minval = jnp.min(dist, axis=1, keepdims=True)
    t = minval - dist
    e = jnp.exp(t)
    s = jnp.sum(e, axis=1, keepdims=True)
    prob_ref[...] = e * (1.0 / s)
    idxf = jnp.min(jnp.where(t == 0.0, iota_ref[...],
                             jnp.float32(2 ** 24)), axis=1)
    idx_ref[...] = idxf.astype(jnp.int32)[None, None, :]

    dist_p = csq + mm2[tn:]
    minval_p = jnp.min(dist_p, axis=1, keepdims=True)
    e_p = jnp.exp(minval_p - dist_p)
    s_p = jnp.sum(e_p, axis=1, keepdims=True)
    pprob_ref[...] = e_p * (1.0 / s_p)


def _vq_pallas(z_flat, zp_flat, codebook, csq):
    n, d = z_flat.shape
    k = codebook.shape[0]
    grid = (n // _TN,)
    iota_row = jnp.arange(k, dtype=jnp.float32)[None, :]
    return pl.pallas_call(
        _vq_body,
        grid=grid,
        in_specs=[
            pl.BlockSpec((_TN, d), lambda i: (i, 0)),
            pl.BlockSpec((_TN, d), lambda i: (i, 0)),
            pl.BlockSpec((k, d), lambda i: (0, 0)),
            pl.BlockSpec((1, k), lambda i: (0, 0)),
            pl.BlockSpec((1, k), lambda i: (0, 0)),
        ],
        out_specs=[
            pl.BlockSpec((_TN, k), lambda i: (i, 0)),
            pl.BlockSpec((_TN, k), lambda i: (i, 0)),
            pl.BlockSpec((1, 1, _TN), lambda i: (i, 0, 0)),
        ],
        out_shape=[
            jax.ShapeDtypeStruct((n, k), jnp.float32),
            jax.ShapeDtypeStruct((n, k), jnp.float32),
            jax.ShapeDtypeStruct((n // _TN, 1, _TN), jnp.int32),
        ],
        compiler_params=pltpu.CompilerParams(
            dimension_semantics=("arbitrary",)),
    )(z_flat, zp_flat, codebook, csq, iota_row)


def _sc_gather(table, idx):
    n = idx.shape[0]
    d = table.shape[1]
    info = plsc.get_sparse_core_info()
    nw = info.num_cores * info.num_subcores
    b_per_w = n // nw
    ch = 96
    nch = b_per_w // ch
    mesh = plsc.VectorSubcoreMesh(core_axis_name="c", subcore_axis_name="s")

    @functools.partial(
        pl.kernel, mesh=mesh,
        out_type=jax.ShapeDtypeStruct((n, d), jnp.float32),
        scratch_types=[
            pltpu.VMEM((b_per_w,), jnp.int32),
            pltpu.VMEM((ch, d), jnp.float32),
            pltpu.VMEM((ch, d), jnp.float32),
            pltpu.SemaphoreType.DMA,
            pltpu.SemaphoreType.DMA,
            pltpu.SemaphoreType.DMA,
            pltpu.SemaphoreType.DMA,
        ],
    )
    def gath(table_hbm, idx_hbm, out_hbm, idx_v, rows0, rows1, g0, g1, o0, o1):
        wid = lax.axis_index("s") * info.num_cores + lax.axis_index("c")
        base = wid * b_per_w
        pltpu.sync_copy(idx_hbm.at[pl.ds(base, b_per_w)], idx_v)
        rows = (rows0, rows1)
        gsem = (g0, g1)
        osem = (o0, o1)
        g = [None] * nch
        o = [None] * nch
        for j in range(nch):
            b = j % 2
            if j >= 2:
                o[j - 2].wait()
            g[j] = pltpu.async_copy(
                table_hbm.at[idx_v.at[pl.ds(j * ch, ch)]], rows[b], gsem[b])
            if j >= 1:
                pb = (j - 1) % 2
                g[j - 1].wait()
                o[j - 1] = pltpu.async_copy(
                    rows[pb], out_hbm.at[pl.ds(base + (j - 1) * ch, ch)],
                    osem[pb])
        g[nch - 1].wait()
        o[nch - 1] = pltpu.async_copy(
            rows[(nch - 1) % 2],
            out_hbm.at[pl.ds(base + (nch - 1) * ch, ch)], osem[(nch - 1) % 2])
        o[nch - 2].wait()
        o[nch - 1].wait()

    return gath(table, idx)


def kernel(z, z_pos, codebook):
    b, d, h, w = z.shape
    n = b * h * w
    z_flat = jnp.transpose(z, (0, 2, 3, 1)).reshape(n, d)
    zp_flat = jnp.transpose(z_pos, (0, 2, 3, 1)).reshape(n, d)
    csq = _csq_pallas(codebook)
    prob, pprob, idx = _vq_pallas(z_flat, zp_flat, codebook, csq)
    idx = idx.reshape(n)
    zq_flat = _sc_gather(codebook, idx)
    z_q = jnp.transpose(zq_flat.reshape(b, h, w, d), (0, 3, 1, 2))
    return z_q, idx, prob, pprob
